# baseline (speedup 1.0000x reference)
import functools

import jax
import jax.numpy as jnp
from jax import lax
from jax.experimental import pallas as pl
from jax.experimental.pallas import tpu as pltpu
from jax.experimental.pallas import tpu_sc as plsc

NW = 32
BLK_CELLS = 20


def _sc_gather_sum(members_flat, table_bf, cell0, cells, K, D):
    blk0 = cell0 // BLK_CELLS
    nblk = blk0 + cells // BLK_CELLS
    idx_per_blk = BLK_CELLS * K
    nb_max = (nblk - blk0 + NW - 1) // NW
    mesh = plsc.VectorSubcoreMesh(core_axis_name="c", subcore_axis_name="s")

    nb2 = (nb_max + 1) // 2

    @functools.partial(
        pl.kernel,
        out_type=jax.ShapeDtypeStruct((cells, D), jnp.float32),
        mesh=mesh,
        scratch_types=[
            pltpu.VMEM((idx_per_blk,), jnp.int32),
            pltpu.VMEM((idx_per_blk,), jnp.int32),
            pltpu.VMEM((idx_per_blk, D), jnp.bfloat16),
            pltpu.VMEM((idx_per_blk, D), jnp.bfloat16),
            pltpu.VMEM((BLK_CELLS, D), jnp.float32),
            pltpu.VMEM((BLK_CELLS, D), jnp.float32),
            pltpu.SemaphoreType.DMA,
            pltpu.SemaphoreType.DMA,
            pltpu.SemaphoreType.DMA,
            pltpu.SemaphoreType.DMA,
            pltpu.SemaphoreType.DMA,
            pltpu.SemaphoreType.DMA,
        ],
        compiler_params=pltpu.CompilerParams(use_tc_tiling_on_sc=False,
                                             needs_layout_passes=False),
    )
    def sc_kernel(members_hbm, nf_hbm, out_hbm,
                  idx0, idx1, rows0, rows1, outv0, outv1,
                  si0, si1, sr0, sr1, so0, so1):
        idx = (idx0, idx1)
        rows = (rows0, rows1)
        outv = (outv0, outv1)
        si = (si0, si1)
        sr = (sr0, sr1)
        so = (so0, so1)
        wid = lax.axis_index("s") * 2 + lax.axis_index("c")

        def t_of(b):
            return blk0 + wid + b * NW

        def idx_fetch(b, u):
            t = t_of(b)

            @pl.when(t < nblk)
            def _():
                pltpu.async_copy(
                    members_hbm.at[pl.ds(t * idx_per_blk, idx_per_blk)],
                    idx[u], si[u])

        def gather_issue(b, u):
            t = t_of(b)

            @pl.when(t < nblk)
            def _():
                pltpu.make_async_copy(
                    members_hbm.at[pl.ds(t * idx_per_blk, idx_per_blk)],
                    idx[u], si[u]).wait()
                pltpu.async_copy(nf_hbm.at[idx[u]], rows[u], sr[u])

        def compute(u):
            for i in range(BLK_CELLS):
                for k in range(D // 32):
                    sl = pl.ds(32 * k, 32)
                    acc = rows[u][K * i, sl]
                    for j in range(1, K):
                        acc = acc + rows[u][K * i + j, sl]
                    lo, hi = plsc.unpack(acc,
                                         format=plsc.PackFormat.INTERLEAVED)
                    outv[u][i, pl.ds(16 * k, 16)] = lo
                    outv[u][i, pl.ds(D // 2 + 16 * k, 16)] = hi

        def phase(b, u):
            t = t_of(b)

            @pl.when(t < nblk)
            def _():
                pltpu.make_async_copy(nf_hbm.at[idx[u]], rows[u], sr[u]).wait()

            gather_issue(b + 1, 1 - u)
            idx_fetch(b + 2, u)

            @pl.when(jnp.logical_and(b >= 2, t < nblk))
            def _():
                pltpu.make_async_copy(
                    outv[u],
                    out_hbm.at[pl.ds(t * BLK_CELLS - cell0, BLK_CELLS), :],
                    so[u]).wait()

            @pl.when(t < nblk)
            def _():
                compute(u)
                pltpu.async_copy(
                    outv[u],
                    out_hbm.at[pl.ds(t * BLK_CELLS - cell0, BLK_CELLS), :],
                    so[u])

        idx_fetch(0, 0)
        gather_issue(0, 0)
        idx_fetch(1, 1)

        def body(bb, carry):
            phase(2 * bb, 0)
            phase(2 * bb + 1, 1)
            return carry

        lax.fori_loop(0, nb2, body, 0)

        for u in (0, 1):
            pltpu.make_async_copy(
                outv[u],
                out_hbm.at[pl.ds(t_of(u) * BLK_CELLS - cell0, BLK_CELLS), :],
                so[u]).wait()

    return sc_kernel(members_flat, table_bf)


def _tc_mlp(x, dims_col, row0, M0, A1, B1, A2, B2, cbias, rows, D):
    BM = 1000
    grid = rows // BM
    boff = row0 // BM

    def tc_kernel(x_ref, d_ref, m0_ref, a1_ref, b1_ref, a2_ref, b2_ref,
                  cb_ref, o_ref):
        x = x_ref[...].astype(jnp.float32)
        cb = cb_ref[...]
        t0 = jnp.dot(x, m0_ref[...], preferred_element_type=jnp.float32) + cb[0:1]
        h1 = jnp.maximum(
            jnp.dot(x, a1_ref[...], preferred_element_type=jnp.float32) + cb[1:2], 0.0)
        t1 = jnp.dot(h1, b1_ref[...], preferred_element_type=jnp.float32) + cb[3:4]
        h2 = jnp.maximum(
            jnp.dot(x, a2_ref[...], preferred_element_type=jnp.float32) + cb[2:3], 0.0)
        t2 = jnp.dot(h2, b2_ref[...], preferred_element_type=jnp.float32) + cb[4:5]
        d = d_ref[...]
        o_ref[...] = jnp.where(d == 0, t0, jnp.where(d == 1, t1, t2))

    wspec = pl.BlockSpec((D, D), lambda i: (0, 0))
    return pl.pallas_call(
        tc_kernel,
        grid=(grid,),
        in_specs=[
            pl.BlockSpec((BM, D), lambda i: (i, 0)),
            pl.BlockSpec((BM, 1), lambda i: (i + boff, 0)),
            wspec, wspec, wspec, wspec, wspec,
            pl.BlockSpec((8, D), lambda i: (0, 0)),
        ],
        out_specs=pl.BlockSpec((BM, D), lambda i: (i, 0)),
        out_shape=jax.ShapeDtypeStruct((rows, D), jnp.float32),
    )(x, dims_col, M0, A1, B1, A2, B2, cbias)


def kernel(node_features, cell_members, cell_dims,
           W0, b0, W1a, b1a, W1b, b1b, W2a, b2a, W2b, b2b, Wout, bout):
    N, D = node_features.shape
    C, K = cell_members.shape
    H = W0.shape[1]

    members_flat = cell_members.astype(jnp.int32).reshape(C * K)
    dims_col = cell_dims.astype(jnp.int32).reshape(C, 1)

    inv_k = jnp.float32(1.0 / K)
    W1s = W1a[:H] + W1a[H:]
    W2s = W2a[:H] + W2a[H:]
    M0 = (W0 @ Wout) * inv_k
    A1 = (W0 @ W1s) * inv_k
    A2 = (W0 @ W2s) * inv_k
    B1 = W1b @ Wout
    B2 = W2b @ Wout
    c0 = b0 @ Wout + bout
    c1 = b0 @ W1s + b1a
    c2 = b0 @ W2s + b2a
    d1 = b1b @ Wout + bout
    d2 = b2b @ Wout + bout
    cbias = jnp.zeros((8, D), jnp.float32)
    cbias = cbias.at[0].set(c0).at[1].set(c1).at[2].set(c2).at[3].set(d1).at[4].set(d2)

    perm = jnp.concatenate([jnp.arange(0, D, 2), jnp.arange(1, D, 2)])
    table_bf = node_features.astype(jnp.bfloat16)
    NCH = 2
    cc = C // NCH
    M0p, A1p, A2p = M0[perm], A1[perm], A2[perm]
    outs = []
    for c in range(NCH):
        x_c = _sc_gather_sum(members_flat, table_bf, c * cc, cc, K, D)
        outs.append(_tc_mlp(x_c, dims_col, c * cc, M0p, A1p, B1, A2p, B2,
                            cbias, cc, D))
    return jnp.concatenate(outs, axis=0)


# speedup vs baseline: 1.0860x; 1.0860x over previous
import functools

import jax
import jax.numpy as jnp
from jax import lax
from jax.experimental import pallas as pl
from jax.experimental.pallas import tpu as pltpu
from jax.experimental.pallas import tpu_sc as plsc

NW = 32
BLK_CELLS = 20


def _sc_gather_sum(members_flat, table_bf, cell0, cells, K, D):
    blk0 = cell0 // BLK_CELLS
    nblk = blk0 + cells // BLK_CELLS
    idx_per_blk = BLK_CELLS * K
    nb_max = (nblk - blk0 + NW - 1) // NW
    mesh = plsc.VectorSubcoreMesh(core_axis_name="c", subcore_axis_name="s")

    nb2 = (nb_max + 1) // 2

    @functools.partial(
        pl.kernel,
        out_type=jax.ShapeDtypeStruct((cells, D), jnp.float32),
        mesh=mesh,
        scratch_types=[
            pltpu.VMEM((idx_per_blk,), jnp.int32),
            pltpu.VMEM((idx_per_blk,), jnp.int32),
            pltpu.VMEM((idx_per_blk, D), jnp.bfloat16),
            pltpu.VMEM((idx_per_blk, D), jnp.bfloat16),
            pltpu.VMEM((BLK_CELLS, D), jnp.float32),
            pltpu.VMEM((BLK_CELLS, D), jnp.float32),
            pltpu.SemaphoreType.DMA,
            pltpu.SemaphoreType.DMA,
            pltpu.SemaphoreType.DMA,
            pltpu.SemaphoreType.DMA,
            pltpu.SemaphoreType.DMA,
            pltpu.SemaphoreType.DMA,
        ],
        compiler_params=pltpu.CompilerParams(use_tc_tiling_on_sc=False,
                                             needs_layout_passes=False),
    )
    def sc_kernel(members_hbm, nf_hbm, out_hbm,
                  idx0, idx1, rows0, rows1, outv0, outv1,
                  si0, si1, sr0, sr1, so0, so1):
        idx = (idx0, idx1)
        rows = (rows0, rows1)
        outv = (outv0, outv1)
        si = (si0, si1)
        sr = (sr0, sr1)
        so = (so0, so1)
        wid = lax.axis_index("s") * 2 + lax.axis_index("c")

        def t_of(b):
            return blk0 + wid + b * NW

        def idx_fetch(b, u):
            t = t_of(b)

            @pl.when(t < nblk)
            def _():
                pltpu.async_copy(
                    members_hbm.at[pl.ds(t * idx_per_blk, idx_per_blk)],
                    idx[u], si[u])

        def gather_issue(b, u):
            t = t_of(b)

            @pl.when(t < nblk)
            def _():
                pltpu.make_async_copy(
                    members_hbm.at[pl.ds(t * idx_per_blk, idx_per_blk)],
                    idx[u], si[u]).wait()
                pltpu.async_copy(nf_hbm.at[idx[u]], rows[u], sr[u])

        def compute(u):
            for i in range(BLK_CELLS):
                for k in range(D // 32):
                    sl = pl.ds(32 * k, 32)
                    lo, hi = plsc.unpack(
                        rows[u][K * i, sl],
                        format=plsc.PackFormat.INTERLEAVED)
                    for j in range(1, K):
                        l2, h2 = plsc.unpack(
                            rows[u][K * i + j, sl],
                            format=plsc.PackFormat.INTERLEAVED)
                        lo = lo + l2
                        hi = hi + h2
                    outv[u][i, pl.ds(16 * k, 16)] = lo
                    outv[u][i, pl.ds(D // 2 + 16 * k, 16)] = hi

        def phase(b, u):
            t = t_of(b)

            @pl.when(t < nblk)
            def _():
                pltpu.make_async_copy(nf_hbm.at[idx[u]], rows[u], sr[u]).wait()

            gather_issue(b + 1, 1 - u)
            idx_fetch(b + 2, u)

            @pl.when(jnp.logical_and(b >= 2, t < nblk))
            def _():
                pltpu.make_async_copy(
                    outv[u],
                    out_hbm.at[pl.ds(t * BLK_CELLS - cell0, BLK_CELLS), :],
                    so[u]).wait()

            @pl.when(t < nblk)
            def _():
                compute(u)
                pltpu.async_copy(
                    outv[u],
                    out_hbm.at[pl.ds(t * BLK_CELLS - cell0, BLK_CELLS), :],
                    so[u])

        idx_fetch(0, 0)
        gather_issue(0, 0)
        idx_fetch(1, 1)

        def body(bb, carry):
            phase(2 * bb, 0)
            phase(2 * bb + 1, 1)
            return carry

        lax.fori_loop(0, nb2, body, 0)

        for u in (0, 1):
            pltpu.make_async_copy(
                outv[u],
                out_hbm.at[pl.ds(t_of(u) * BLK_CELLS - cell0, BLK_CELLS), :],
                so[u]).wait()

    return sc_kernel(members_flat, table_bf)


def _tc_mlp(x, dims_col, row0, M0, A1, B1, A2, B2, cbias, rows, D):
    BM = 1000
    grid = rows // BM
    boff = row0 // BM

    def tc_kernel(x_ref, d_ref, m0_ref, a1_ref, b1_ref, a2_ref, b2_ref,
                  cb_ref, o_ref):
        x = x_ref[...].astype(jnp.float32)
        cb = cb_ref[...]
        t0 = jnp.dot(x, m0_ref[...], preferred_element_type=jnp.float32) + cb[0:1]
        h1 = jnp.maximum(
            jnp.dot(x, a1_ref[...], preferred_element_type=jnp.float32) + cb[1:2], 0.0)
        t1 = jnp.dot(h1, b1_ref[...], preferred_element_type=jnp.float32) + cb[3:4]
        h2 = jnp.maximum(
            jnp.dot(x, a2_ref[...], preferred_element_type=jnp.float32) + cb[2:3], 0.0)
        t2 = jnp.dot(h2, b2_ref[...], preferred_element_type=jnp.float32) + cb[4:5]
        d = d_ref[...].reshape(1, BM).T
        o_ref[...] = jnp.where(d == 0, t0, jnp.where(d == 1, t1, t2))

    wspec = pl.BlockSpec((D, D), lambda i: (0, 0))
    return pl.pallas_call(
        tc_kernel,
        grid=(grid,),
        in_specs=[
            pl.BlockSpec((BM, D), lambda i: (i, 0)),
            pl.BlockSpec((1, 1, BM), lambda i: (i + boff, 0, 0)),
            wspec, wspec, wspec, wspec, wspec,
            pl.BlockSpec((8, D), lambda i: (0, 0)),
        ],
        out_specs=pl.BlockSpec((BM, D), lambda i: (i, 0)),
        out_shape=jax.ShapeDtypeStruct((rows, D), jnp.float32),
    )(x, dims_col, M0, A1, B1, A2, B2, cbias)


def kernel(node_features, cell_members, cell_dims,
           W0, b0, W1a, b1a, W1b, b1b, W2a, b2a, W2b, b2b, Wout, bout):
    N, D = node_features.shape
    C, K = cell_members.shape
    H = W0.shape[1]

    members_flat = cell_members.astype(jnp.int32).reshape(C * K)
    dims_col = cell_dims.astype(jnp.int32).reshape(C // 1000, 1, 1000)

    inv_k = jnp.float32(1.0 / K)
    W1s = W1a[:H] + W1a[H:]
    W2s = W2a[:H] + W2a[H:]
    M0 = (W0 @ Wout) * inv_k
    A1 = (W0 @ W1s) * inv_k
    A2 = (W0 @ W2s) * inv_k
    B1 = W1b @ Wout
    B2 = W2b @ Wout
    c0 = b0 @ Wout + bout
    c1 = b0 @ W1s + b1a
    c2 = b0 @ W2s + b2a
    d1 = b1b @ Wout + bout
    d2 = b2b @ Wout + bout
    cbias = jnp.zeros((8, D), jnp.float32)
    cbias = cbias.at[0].set(c0).at[1].set(c1).at[2].set(c2).at[3].set(d1).at[4].set(d2)

    perm = jnp.concatenate([jnp.arange(0, D, 2), jnp.arange(1, D, 2)])
    table_bf = node_features.astype(jnp.bfloat16)
    NCH = 2
    cc = C // NCH
    M0p, A1p, A2p = M0[perm], A1[perm], A2[perm]
    outs = []
    for c in range(NCH):
        x_c = _sc_gather_sum(members_flat, table_bf, c * cc, cc, K, D)
        outs.append(_tc_mlp(x_c, dims_col, c * cc, M0p, A1p, B1, A2p, B2,
                            cbias, cc, D))
    return jnp.concatenate(outs, axis=0)
